# baseline (speedup 1.0000x reference)
import jax
import jax.numpy as jnp
from jax.experimental import pallas as pl

_T = 0.07
_TOP_K = 64
_NUM_CLASSES = 100
_K_QUEUE = 1000


def _mlp_body(x_ref, w1_ref, b1_ref, w2_ref, b2_ref, out_ref):
    h = jnp.dot(x_ref[...], w1_ref[...], preferred_element_type=jnp.float32)
    h = jnp.maximum(h + b1_ref[...], 0.0)
    out_ref[...] = jnp.dot(h, w2_ref[...], preferred_element_type=jnp.float32) + b2_ref[...]


def kernel(x_q, y_batch, W1, b1, W2, b2, feature_queues):
    B, D = x_q.shape
    C = W2.shape[1]
    logits = pl.pallas_call(
        _mlp_body,
        out_shape=jax.ShapeDtypeStruct((B, C), jnp.float32),
    )(x_q, W1, b1[None, :], W2, b2[None, :])

    q_proj = x_q / jnp.linalg.norm(x_q, axis=1, keepdims=True)
    all_features = feature_queues.reshape(-1, feature_queues.shape[-1])
    all_labels = jnp.repeat(jnp.arange(_NUM_CLASSES), _K_QUEUE)
    cos_sim = q_proj @ all_features.T
    pos_mask = all_labels[None, :] == y_batch[:, None]
    neg_inf = jnp.float32(-jnp.inf)
    pos_sim = jnp.where(pos_mask, cos_sim, neg_inf)
    pos_score = jnp.max(pos_sim, axis=1, keepdims=True)
    neg_sim = jnp.where(~pos_mask, cos_sim, neg_inf)
    neg_score, _ = jax.lax.top_k(neg_sim, _TOP_K)
    logits_con = jnp.concatenate([pos_score, neg_score], axis=1) / _T
    return (logits, logits_con)


# speedup vs baseline: 6.9093x; 6.9093x over previous
import jax
import jax.numpy as jnp
from jax.experimental import pallas as pl

_T = 0.07
_TOP_K = 64
_NUM_CLASSES = 100
_K_QUEUE = 1000
_TILE = 512
_G = 64
_CPT = _TILE // _G


def _head_body(x_ref, w1_ref, b1_ref, w2_ref, b2_ref, logits_ref, qproj_ref):
    x = x_ref[...]
    h = jnp.dot(x, w1_ref[...], preferred_element_type=jnp.float32)
    h = jnp.maximum(h + b1_ref[...], 0.0)
    logits_ref[...] = jnp.dot(h, w2_ref[...], preferred_element_type=jnp.float32) + b2_ref[...]
    nrm = jnp.sum(x * x, axis=1, keepdims=True)
    qproj_ref[...] = x * jax.lax.rsqrt(nrm)


def _sim_body(y_ref, q_ref, f_ref, cos_ref, cmax_ref, pos_ref, *, n_total):
    j = pl.program_id(0)
    s = jax.lax.dot_general(q_ref[...], f_ref[...], (((1,), (1,)), ((), ())),
                            preferred_element_type=jnp.float32)
    cols = j * _TILE + jax.lax.broadcasted_iota(jnp.int32, s.shape, 1)
    valid = cols < n_total
    cls = cols // _K_QUEUE
    pos = jnp.logical_and(cls == y_ref[...], valid)
    neg_inf = jnp.float32(-jnp.inf)
    negval = jnp.where(jnp.logical_or(pos, jnp.logical_not(valid)), neg_inf, s)
    posval = jnp.where(pos, s, neg_inf)
    cos_ref[...] = negval
    parts = [jnp.max(negval[:, c * _G:(c + 1) * _G], axis=1, keepdims=True)
             for c in range(_CPT)]
    cmax_ref[...] = jnp.concatenate(parts, axis=1)[None]
    pmax = jnp.max(posval, axis=1, keepdims=True)

    @pl.when(j == 0)
    def _():
        pos_ref[...] = pmax

    @pl.when(j > 0)
    def _():
        pos_ref[...] = jnp.maximum(pos_ref[...], pmax)


def kernel(x_q, y_batch, W1, b1, W2, b2, feature_queues):
    B, D = x_q.shape
    C = W2.shape[1]
    n_total = feature_queues.shape[0] * feature_queues.shape[1]
    all_features = feature_queues.reshape(n_total, D)
    n_tiles = (n_total + _TILE - 1) // _TILE
    n_pad = n_tiles * _TILE
    n_chunks = n_pad // _G

    logits, q_proj = pl.pallas_call(
        _head_body,
        out_shape=(jax.ShapeDtypeStruct((B, C), jnp.float32),
                   jax.ShapeDtypeStruct((B, D), jnp.float32)),
    )(x_q, W1, b1[None, :], W2, b2[None, :])

    import functools
    cos, cmax, pos_score = pl.pallas_call(
        functools.partial(_sim_body, n_total=n_total),
        grid=(n_tiles,),
        in_specs=[
            pl.BlockSpec((B, 1), lambda j: (0, 0)),
            pl.BlockSpec((B, D), lambda j: (0, 0)),
            pl.BlockSpec((_TILE, D), lambda j: (j, 0)),
        ],
        out_specs=[
            pl.BlockSpec((B, _TILE), lambda j: (0, j)),
            pl.BlockSpec((1, B, _CPT), lambda j: (j, 0, 0)),
            pl.BlockSpec((B, 1), lambda j: (0, 0)),
        ],
        out_shape=[
            jax.ShapeDtypeStruct((B, n_pad), jnp.float32),
            jax.ShapeDtypeStruct((n_tiles, B, _CPT), jnp.float32),
            jax.ShapeDtypeStruct((B, 1), jnp.float32),
        ],
    )(y_batch.astype(jnp.int32)[:, None], q_proj, all_features)

    cmax = cmax.transpose(1, 0, 2).reshape(B, n_chunks)
    _, chunk_idx = jax.lax.top_k(cmax, _TOP_K)
    cos3 = cos.reshape(B, n_chunks, _G)
    pool = jnp.take_along_axis(cos3, chunk_idx[:, :, None], axis=1)
    neg_score, _ = jax.lax.top_k(pool.reshape(B, _TOP_K * _G), _TOP_K)
    logits_con = jnp.concatenate([pos_score, neg_score], axis=1) / _T
    return (logits, logits_con)


# speedup vs baseline: 9.6223x; 1.3927x over previous
import jax
import jax.numpy as jnp
from jax.experimental import pallas as pl

_T = 0.07
_TOP_K = 64
_NUM_CLASSES = 100
_K_QUEUE = 1000
_TILE = 512
_G = 64
_CPT = _TILE // _G


def _head_body(x_ref, w1_ref, b1_ref, w2_ref, b2_ref, logits_ref, qproj_ref):
    x = x_ref[...]
    h = jnp.dot(x, w1_ref[...], preferred_element_type=jnp.float32)
    h = jnp.maximum(h + b1_ref[...], 0.0)
    logits_ref[...] = jnp.dot(h, w2_ref[...], preferred_element_type=jnp.float32) + b2_ref[...]
    nrm = jnp.sum(x * x, axis=1, keepdims=True)
    qproj_ref[...] = x * jax.lax.rsqrt(nrm)


def _sim_body(y_ref, q_ref, f_ref, cos_ref, cmax_ref, pos_ref, *, n_total):
    j = pl.program_id(0)
    s = jax.lax.dot_general(q_ref[...], f_ref[...], (((1,), (1,)), ((), ())),
                            preferred_element_type=jnp.float32)
    cols = j * _TILE + jax.lax.broadcasted_iota(jnp.int32, s.shape, 1)
    valid = cols < n_total
    cls = cols // _K_QUEUE
    pos = jnp.logical_and(cls == y_ref[...], valid)
    neg_inf = jnp.float32(-jnp.inf)
    negval = jnp.where(jnp.logical_or(pos, jnp.logical_not(valid)), neg_inf, s)
    posval = jnp.where(pos, s, neg_inf)
    cos_ref[...] = negval
    parts = [jnp.max(negval[:, c * _G:(c + 1) * _G], axis=1, keepdims=True)
             for c in range(_CPT)]
    cmax_ref[...] = jnp.concatenate(parts, axis=1)[None]
    pmax = jnp.max(posval, axis=1, keepdims=True)

    @pl.when(j == 0)
    def _():
        pos_ref[...] = pmax

    @pl.when(j > 0)
    def _():
        pos_ref[...] = jnp.maximum(pos_ref[...], pmax)


def _lane_iota(shape):
    return jax.lax.broadcasted_iota(jnp.int32, shape, len(shape) - 1)


def _xor_shuffle(x, j):
    L = x.shape[-1]
    xl = jnp.concatenate([x[:, j:], x[:, :j]], axis=1)
    xr = jnp.concatenate([x[:, L - j:], x[:, :L - j]], axis=1)
    return jnp.where((_lane_iota(x.shape) & j) == 0, xl, xr)


def _ce(v, idx, j, keep_max):
    pv = _xor_shuffle(v, j)
    nv = jnp.where(keep_max, jnp.maximum(v, pv), jnp.minimum(v, pv))
    if idx is None:
        return nv, None
    pidx = _xor_shuffle(idx, j)
    take_p = (keep_max & (pv > v)) | (jnp.logical_not(keep_max) & (pv < v))
    return nv, jnp.where(take_p, pidx, idx)


def _merge64(v, idx, desc):
    i = _lane_iota(v.shape)
    for j in (32, 16, 8, 4, 2, 1):
        keep_max = jnp.logical_not(((i & j) == 0) ^ desc)
        v, idx = _ce(v, idx, j, keep_max)
    return v, idx


def _sort64(v, idx, desc):
    i = _lane_iota(v.shape)
    for k in (2, 4, 8, 16, 32):
        desc_k = ((i & k) != 0) ^ desc
        j = k // 2
        while j:
            keep_max = jnp.logical_not(((i & j) == 0) ^ desc_k)
            v, idx = _ce(v, idx, j, keep_max)
            j //= 2
    return _merge64(v, idx, desc)


def _top64(v, idx):
    W = v.shape[-1]
    desc = (_lane_iota(v.shape) & (W // 2)) == 0
    v, idx = _sort64(v, idx, desc)
    H = W // 2
    while H >= 64:
        a, b = v[:, :H], v[:, H:]
        if idx is not None:
            idx = jnp.where(b > a, idx[:, H:], idx[:, :H])
        v = jnp.maximum(a, b)
        i = _lane_iota(v.shape)
        desc = ((i & (H // 2)) == 0) if H > 64 else (i >= 0)
        v, idx = _merge64(v, idx, desc)
        H //= 2
    return v, idx


def _select_body(cmax_ref, idx_ref):
    v = cmax_ref[...]
    R, n = v.shape
    W = 1 << (n - 1).bit_length()
    if W > n:
        v = jnp.concatenate(
            [v, jnp.full((R, W - n), -jnp.inf, jnp.float32)], axis=1)
    _, idx = _top64(v, _lane_iota(v.shape))
    idx_ref[...] = idx


def _final_body(pool_ref, pos_ref, out_ref):
    neg, _ = _top64(pool_ref[...], None)
    out_ref[...] = jnp.concatenate([pos_ref[...], neg], axis=1) / _T


def kernel(x_q, y_batch, W1, b1, W2, b2, feature_queues):
    B, D = x_q.shape
    C = W2.shape[1]
    n_total = feature_queues.shape[0] * feature_queues.shape[1]
    all_features = feature_queues.reshape(n_total, D)
    n_tiles = (n_total + _TILE - 1) // _TILE
    n_pad = n_tiles * _TILE
    n_chunks = n_pad // _G

    logits, q_proj = pl.pallas_call(
        _head_body,
        out_shape=(jax.ShapeDtypeStruct((B, C), jnp.float32),
                   jax.ShapeDtypeStruct((B, D), jnp.float32)),
    )(x_q, W1, b1[None, :], W2, b2[None, :])

    import functools
    cos, cmax, pos_score = pl.pallas_call(
        functools.partial(_sim_body, n_total=n_total),
        grid=(n_tiles,),
        in_specs=[
            pl.BlockSpec((B, 1), lambda j: (0, 0)),
            pl.BlockSpec((B, D), lambda j: (0, 0)),
            pl.BlockSpec((_TILE, D), lambda j: (j, 0)),
        ],
        out_specs=[
            pl.BlockSpec((B, _TILE), lambda j: (0, j)),
            pl.BlockSpec((1, B, _CPT), lambda j: (j, 0, 0)),
            pl.BlockSpec((B, 1), lambda j: (0, 0)),
        ],
        out_shape=[
            jax.ShapeDtypeStruct((B, n_pad), jnp.float32),
            jax.ShapeDtypeStruct((n_tiles, B, _CPT), jnp.float32),
            jax.ShapeDtypeStruct((B, 1), jnp.float32),
        ],
    )(y_batch.astype(jnp.int32)[:, None], q_proj, all_features)

    cmax = cmax.transpose(1, 0, 2).reshape(B, n_chunks)
    RB = 256
    chunk_idx = pl.pallas_call(
        _select_body,
        grid=(B // RB,),
        in_specs=[pl.BlockSpec((RB, n_chunks), lambda j: (j, 0))],
        out_specs=pl.BlockSpec((RB, _TOP_K), lambda j: (j, 0)),
        out_shape=jax.ShapeDtypeStruct((B, _TOP_K), jnp.int32),
    )(cmax)

    cos3 = cos.reshape(B, n_chunks, _G)
    pool = jnp.take_along_axis(cos3, chunk_idx[:, :, None], axis=1)

    logits_con = pl.pallas_call(
        _final_body,
        grid=(B // RB,),
        in_specs=[pl.BlockSpec((RB, _TOP_K * _G), lambda j: (j, 0)),
                  pl.BlockSpec((RB, 1), lambda j: (j, 0))],
        out_specs=pl.BlockSpec((RB, 1 + _TOP_K), lambda j: (j, 0)),
        out_shape=jax.ShapeDtypeStruct((B, 1 + _TOP_K), jnp.float32),
    )(pool.reshape(B, _TOP_K * _G), pos_score)
    return (logits, logits_con)
